# Initial kernel scaffold; baseline (speedup 1.0000x reference)
import jax
import jax.numpy as jnp
from jax.experimental import pallas as pl


def kernel(edge_index, knowledge_tag_idx, test_id_idx, big_category_idx, user_emb, item_emb, tag_emb, testid_emb, bigcat_emb):
    raise NotImplementedError("write your pallas kernel here")


# speedup vs baseline: 1.2395x; 1.2395x over previous
import functools

import jax
import jax.numpy as jnp
from jax import lax
from jax.experimental import pallas as pl
from jax.experimental.pallas import tpu as pltpu
from jax.experimental.pallas import tpu_sc as plsc

_LAYERS = 3
_CHUNK = 200


@functools.cache
def _sc_gather(num_table_rows: int, num_idx: int, dim: int):
  info = plsc.get_sparse_core_info()
  nw = info.num_cores * info.num_subcores
  b_per_w = num_idx // nw
  assert num_idx % nw == 0 and b_per_w % _CHUNK == 0 and b_per_w % 8 == 0
  n_steps = b_per_w // _CHUNK
  mesh = plsc.VectorSubcoreMesh(core_axis_name="c", subcore_axis_name="s")

  @functools.partial(
      pl.kernel,
      mesh=mesh,
      out_type=jax.ShapeDtypeStruct((num_idx, dim), jnp.float32),
      scratch_types=[
          pltpu.VMEM((_CHUNK,), jnp.int32),
          pltpu.VMEM((_CHUNK, dim), jnp.float32),
          pltpu.SemaphoreType.DMA,
      ],
  )
  def k(table_hbm, idx_hbm, out_hbm, idx_v, rows_v, sem):
    wid = lax.axis_index("s") * info.num_cores + lax.axis_index("c")
    base = wid * b_per_w

    @pl.loop(0, n_steps)
    def _(i):
      off = base + i * _CHUNK
      pltpu.sync_copy(idx_hbm.at[pl.ds(off, _CHUNK)], idx_v)
      pltpu.async_copy(table_hbm.at[idx_v], rows_v, sem).wait()
      pltpu.sync_copy(rows_v, out_hbm.at[pl.ds(off, _CHUNK)])

  return k


def _scale_body(g_ref, n_ref, o_ref):
  o_ref[...] = g_ref[...] * n_ref[...]


def _dot_body(a_ref, b_ref, o_ref):
  o_ref[...] = jnp.sum(a_ref[...] * b_ref[...], axis=1, keepdims=True)


def _scale(gathered, norm2d, blk=8000):
  e, d = gathered.shape
  return pl.pallas_call(
      _scale_body,
      grid=(e // blk,),
      in_specs=[
          pl.BlockSpec((blk, d), lambda i: (i, 0)),
          pl.BlockSpec((blk, 1), lambda i: (i, 0)),
      ],
      out_specs=pl.BlockSpec((blk, d), lambda i: (i, 0)),
      out_shape=jax.ShapeDtypeStruct((e, d), jnp.float32),
  )(gathered, norm2d)


def _dot(a, b, blk=8000):
  e, d = a.shape
  return pl.pallas_call(
      _dot_body,
      grid=(e // blk,),
      in_specs=[
          pl.BlockSpec((blk, d), lambda i: (i, 0)),
          pl.BlockSpec((blk, d), lambda i: (i, 0)),
      ],
      out_specs=pl.BlockSpec((blk, 1), lambda i: (i, 0)),
      out_shape=jax.ShapeDtypeStruct((e, 1), jnp.float32),
  )(a, b)[:, 0]


def kernel(edge_index, knowledge_tag_idx, test_id_idx, big_category_idx,
           user_emb, item_emb, tag_emb, testid_emb, bigcat_emb):
  item_total = (item_emb
                + jnp.take(tag_emb, knowledge_tag_idx, axis=0)
                + jnp.take(testid_emb, test_id_idx, axis=0)
                + jnp.take(bigcat_emb, big_category_idx, axis=0)) / 4.0
  x = jnp.concatenate([user_emb, item_total], axis=0)
  num_nodes, dim = x.shape
  row = edge_index[0]
  col = edge_index[1]
  num_edges = row.shape[0]

  deg = jnp.zeros((num_nodes,), x.dtype).at[col].add(1.0)
  deg_inv_sqrt = jnp.where(deg > 0,
                           1.0 / jnp.sqrt(jnp.where(deg > 0, deg, 1.0)), 0.0)
  norm2d = (deg_inv_sqrt[row] * deg_inv_sqrt[col])[:, None]

  dim = 128
  x = jnp.pad(x, ((0, 0), (0, dim - x.shape[1])))
  gather = _sc_gather(num_nodes, num_edges, dim)

  alpha = 1.0 / (_LAYERS + 1)
  out = x * alpha
  for _ in range(_LAYERS):
    msgs = _scale(gather(x, row), norm2d)
    x = jnp.zeros_like(x).at[col].add(msgs)
    out = out + x * alpha

  return _dot(gather(out, row), gather(out, col))
